# baseline (speedup 1.0000x reference)
import functools
import jax
import jax.numpy as jnp
from jax import lax
from jax.experimental import pallas as pl
from jax.experimental.pallas import tpu as pltpu
from jax.experimental.pallas import tpu_sc as plsc

VOCAB = 1000
NC, NS = 2, 16
NW = NC * NS
CHUNK = 50


def _body(b_per_w, t_chunks, idx_hbm, table_hbm, out_hbm,
          idx_v, buf0, buf1, gs0, gs1, ss0, ss1):
    wid = lax.axis_index("s") * NC + lax.axis_index("c")
    pltpu.sync_copy(idx_hbm.at[wid], idx_v)
    n_chunks = b_per_w * t_chunks

    def g_start(c, buf, sem):
        pltpu.async_copy(table_hbm.at[idx_v.at[c]], buf, sem)

    def g_wait(c, buf, sem):
        pltpu.make_async_copy(table_hbm.at[idx_v.at[c]], buf, sem).wait()

    def out_slice(c):
        b = wid * b_per_w + c // t_chunks
        t0 = (c % t_chunks) * CHUNK
        return out_hbm.at[b, pl.ds(t0, CHUNK)]

    def s_start(c, buf, sem):
        pltpu.async_copy(buf, out_slice(c), sem)

    def s_wait(c, buf, sem):
        pltpu.make_async_copy(buf, out_slice(c), sem).wait()

    g_start(0, buf0, gs0)
    g_start(1, buf1, gs1)

    def it(i, carry):
        c0 = 2 * i
        c1 = c0 + 1
        g_wait(c0, buf0, gs0)
        s_start(c0, buf0, ss0)
        g_wait(c1, buf1, gs1)
        s_start(c1, buf1, ss1)
        s_wait(c0, buf0, ss0)
        g_start(c0 + 2, buf0, gs0)
        s_wait(c1, buf1, ss1)
        g_start(c1 + 2, buf1, gs1)
        return carry

    lax.fori_loop(0, n_chunks // 2 - 1, it, 0)

    c0 = n_chunks - 2
    c1 = n_chunks - 1
    g_wait(c0, buf0, gs0)
    s_start(c0, buf0, ss0)
    g_wait(c1, buf1, gs1)
    s_start(c1, buf1, ss1)
    s_wait(c0, buf0, ss0)
    s_wait(c1, buf1, ss1)


def kernel(context, table):
    b, t = context.shape
    assert b % NW == 0 and t % CHUNK == 0
    b_per_w = b // NW
    t_chunks = t // CHUNK
    n_chunks = b_per_w * t_chunks
    idx = context.reshape(NW, n_chunks, CHUNK).astype(jnp.int32)

    mesh = plsc.VectorSubcoreMesh(core_axis_name="c", subcore_axis_name="s")
    run = pl.kernel(
        functools.partial(_body, b_per_w, t_chunks),
        out_type=jax.ShapeDtypeStruct((b, t, VOCAB), jnp.float32),
        mesh=mesh,
        scratch_types=[
            pltpu.VMEM((n_chunks, CHUNK), jnp.int32),
            pltpu.VMEM((CHUNK, VOCAB), jnp.float32),
            pltpu.VMEM((CHUNK, VOCAB), jnp.float32),
            pltpu.SemaphoreType.DMA,
            pltpu.SemaphoreType.DMA,
            pltpu.SemaphoreType.DMA,
            pltpu.SemaphoreType.DMA,
        ],
        compiler_params=pltpu.CompilerParams(use_tc_tiling_on_sc=False),
    )
    return run(idx, table)


# speedup vs baseline: 1.7233x; 1.7233x over previous
import functools
import jax
import jax.numpy as jnp
from jax import lax
from jax.experimental import pallas as pl
from jax.experimental.pallas import tpu as pltpu
from jax.experimental.pallas import tpu_sc as plsc

VOCAB = 1000
VPAD = 1024
NBLK = VPAD // 128
NC, NS = 2, 16
NW = NC * NS
CHUNK = 32


def _body(n_chunks, idx_hbm, table_hbm, out_hbm,
          idx_v, ic0, ic1, buf0, buf1, gs0, gs1, ss0, ss1):
    wid = lax.axis_index("s") * NC + lax.axis_index("c")
    pltpu.sync_copy(idx_hbm.at[wid], idx_v)
    base = wid * (n_chunks * CHUNK)

    def compute_idx(c, ic):
        for k in range(CHUNK // 16):
            v = idx_v[pl.ds(c * CHUNK + 16 * k, 16)] * 8
            for C in range(NBLK):
                ic[pl.ds(C * CHUNK + 16 * k, 16)] = v + C

    def g_descs(ic, buf, sem):
        return [(table_hbm.at[ic.at[pl.ds(C * CHUNK, CHUNK)]],
                 buf.at[:, pl.ds(128 * C, 128)], sem)
                for C in range(NBLK)]

    def g_start(ic, buf, sem):
        for s, d, sm in g_descs(ic, buf, sem):
            pltpu.async_copy(s, d, sm)

    def g_wait(ic, buf, sem):
        for s, d, sm in g_descs(ic, buf, sem):
            pltpu.make_async_copy(s, d, sm).wait()

    def out_slice(c):
        return out_hbm.at[pl.ds(base + c * CHUNK, CHUNK)]

    def s_start(c, buf, sem):
        pltpu.async_copy(buf, out_slice(c), sem)

    def s_wait(c, buf, sem):
        pltpu.make_async_copy(buf, out_slice(c), sem).wait()

    compute_idx(0, ic0)
    g_start(ic0, buf0, gs0)
    compute_idx(1, ic1)
    g_start(ic1, buf1, gs1)

    def it(i, carry):
        c0 = 2 * i
        c1 = c0 + 1
        g_wait(ic0, buf0, gs0)
        s_start(c0, buf0, ss0)
        g_wait(ic1, buf1, gs1)
        s_start(c1, buf1, ss1)
        s_wait(c0, buf0, ss0)
        compute_idx(c0 + 2, ic0)
        g_start(ic0, buf0, gs0)
        s_wait(c1, buf1, ss1)
        compute_idx(c1 + 2, ic1)
        g_start(ic1, buf1, gs1)
        return carry

    lax.fori_loop(0, n_chunks // 2 - 1, it, 0)

    c0 = n_chunks - 2
    c1 = n_chunks - 1
    g_wait(ic0, buf0, gs0)
    s_start(c0, buf0, ss0)
    g_wait(ic1, buf1, gs1)
    s_start(c1, buf1, ss1)
    s_wait(c0, buf0, ss0)
    s_wait(c1, buf1, ss1)


def kernel(context, table):
    b, t = context.shape
    n = b * t
    assert n % (NW * CHUNK) == 0
    n_chunks = n // (NW * CHUNK)
    assert n_chunks % 2 == 0
    idx = context.reshape(NW, n_chunks * CHUNK).astype(jnp.int32)
    table_r = jnp.pad(table, ((0, 0), (0, VPAD - VOCAB))).reshape(VOCAB * NBLK, 128)

    mesh = plsc.VectorSubcoreMesh(core_axis_name="c", subcore_axis_name="s")
    run = pl.kernel(
        functools.partial(_body, n_chunks),
        out_type=jax.ShapeDtypeStruct((n, VPAD), jnp.float32),
        mesh=mesh,
        scratch_types=[
            pltpu.VMEM((n_chunks * CHUNK,), jnp.int32),
            pltpu.VMEM((NBLK * CHUNK,), jnp.int32),
            pltpu.VMEM((NBLK * CHUNK,), jnp.int32),
            pltpu.VMEM((CHUNK, VPAD), jnp.float32),
            pltpu.VMEM((CHUNK, VPAD), jnp.float32),
            pltpu.SemaphoreType.DMA,
            pltpu.SemaphoreType.DMA,
            pltpu.SemaphoreType.DMA,
            pltpu.SemaphoreType.DMA,
        ],
    )
    out = run(idx, table_r)
    return out[:, :VOCAB].reshape(b, t, VOCAB)
